# baseline (speedup 1.0000x reference)
import functools

import jax
import jax.numpy as jnp
from jax import lax
from jax.experimental import pallas as pl
from jax.experimental.pallas import tpu as pltpu
from jax.experimental.pallas import tpu_sc as plsc

N = 10000
M = 320000
D = 128
HD = 64
NC = 2
NS = 16
EPT = M // NS
EB = 40
NBLK = EPT // EB
NP = 10240
NPT = NP // NS


def _sc_edge_body(write_u, ce, gsrc, ex, srcoff, dst, se, *refs):
    if write_u:
        (u_out, nr_out, sig_out, stats_out,
         src_v, dst_v, se_v, ce_v, g_v, ex_v, sig_v, nr_v, u_v, st_v,
         gsem, esem) = refs
    else:
        (nr_out, sig_out,
         src_v, dst_v, se_v, ce_v, g_v, ex_v, sig_v, nr_v,
         gsem, esem) = refs

    c = lax.axis_index("c")
    s = lax.axis_index("s")
    zero16 = jnp.zeros((16,), jnp.float32)

    def block(j, carry):
        base = s * EPT + j * EB
        pltpu.sync_copy(srcoff.at[pl.ds(c * M + base, EB)], src_v)
        pltpu.sync_copy(dst.at[pl.ds(base, EB)], dst_v)
        pltpu.sync_copy(se.at[pl.ds(base, EB)], se_v.at[pl.ds(0, EB)])
        pltpu.sync_copy(ce.at[c, pl.ds(base, EB)], ce_v)
        gcp = pltpu.async_copy(gsrc.at[src_v], g_v, gsem)
        ecp = pltpu.async_copy(ex.at[dst_v], ex_v, esem)
        gcp.wait()
        ecp.wait()

        def edge(jj, car):
            ses = se_v[pl.ds(jj, 16)][0]
            new = []
            for k in range(4):
                sl = pl.ds(k * 16, 16)
                e16 = ce_v[jj, sl] + g_v[jj, sl] + ex_v[jj, pl.ds(c * HD + k * 16, 16)]
                sg = 1.0 / (1.0 + jnp.exp(-e16))
                sig_v[jj, sl] = sg
                nr_v[jj, sl] = sg * g_v[jj, pl.ds(HD + k * 16, 16)]
                if write_u:
                    u16 = e16 * ses
                    u_v[jj, sl] = u16
                    new.append(car[k] + u16)
                    new.append(car[4 + k] + u16 * u16)
            if write_u:
                return tuple(new[0::2]) + tuple(new[1::2])
            return car

        carry = lax.fori_loop(0, EB, edge, carry)
        pltpu.sync_copy(nr_v, nr_out.at[c, pl.ds(base, EB)])
        pltpu.sync_copy(sig_v, sig_out.at[c, pl.ds(base, EB)])
        if write_u:
            pltpu.sync_copy(u_v, u_out.at[c, pl.ds(base, EB)])
        return carry

    init = tuple([zero16] * 8) if write_u else 0
    carry = lax.fori_loop(0, NBLK, block, init)

    if write_u:
        for k in range(4):
            st_v[pl.ds(k * 16, 16)] = carry[k]
            st_v[pl.ds(HD + k * 16, 16)] = carry[4 + k]
        pltpu.sync_copy(st_v, stats_out.at[pl.ds((c * NS + s) * D, D)])


def _make_sc_edge(write_u):
    out_type = [
        jax.ShapeDtypeStruct((NC, M, HD), jnp.float32),
        jax.ShapeDtypeStruct((NC, M, HD), jnp.float32),
    ]
    scratch = [
        pltpu.VMEM((EB,), jnp.int32),
        pltpu.VMEM((EB,), jnp.int32),
        pltpu.VMEM((EB + 16,), jnp.float32),
        pltpu.VMEM((EB, HD), jnp.float32),
        pltpu.VMEM((EB, D), jnp.float32),
        pltpu.VMEM((EB, D), jnp.float32),
        pltpu.VMEM((EB, HD), jnp.float32),
        pltpu.VMEM((EB, HD), jnp.float32),
    ]
    if write_u:
        out_type = [jax.ShapeDtypeStruct((NC, M, HD), jnp.float32)] + out_type
        out_type.append(jax.ShapeDtypeStruct((NC * NS * D,), jnp.float32))
        scratch.append(pltpu.VMEM((EB, HD), jnp.float32))
        scratch.append(pltpu.VMEM((D,), jnp.float32))
    scratch += [
        pltpu.SemaphoreType.DMA,
        pltpu.SemaphoreType.DMA,
    ]
    return pl.kernel(
        functools.partial(_sc_edge_body, write_u),
        out_type=out_type,
        mesh=plsc.VectorSubcoreMesh(core_axis_name="c", subcore_axis_name="s"),
        scratch_types=scratch,
    )


_sc_edge_u = _make_sc_edge(True)
_sc_edge_nou = _make_sc_edge(False)


def _sc_scatter_body(nr_all, sig_all, dst, zrows, *refs):
    (num_out, den_out, dst_v, nr_v, sig_v, num_acc, den_acc) = refs
    c = lax.axis_index("c")
    s = lax.axis_index("s")

    pltpu.sync_copy(zrows, num_acc.at[pl.ds(s * NPT, NPT)])
    pltpu.sync_copy(zrows, den_acc.at[pl.ds(s * NPT, NPT)])
    plsc.subcore_barrier()

    def block(j, _):
        base = s * EPT + j * EB
        pltpu.sync_copy(dst.at[pl.ds(base, EB)], dst_v)
        pltpu.sync_copy(nr_all.at[c, pl.ds(base, EB)], nr_v)
        pltpu.sync_copy(sig_all.at[c, pl.ds(base, EB)], sig_v)
        pltpu.sync_copy(nr_v, num_acc.at[dst_v], add=True)
        pltpu.sync_copy(sig_v, den_acc.at[dst_v], add=True)
        return 0

    lax.fori_loop(0, NBLK, block, 0)
    plsc.subcore_barrier()
    pltpu.sync_copy(num_acc.at[pl.ds(s * NPT, NPT)],
                    num_out.at[c, pl.ds(s * NPT, NPT)])
    pltpu.sync_copy(den_acc.at[pl.ds(s * NPT, NPT)],
                    den_out.at[c, pl.ds(s * NPT, NPT)])


_sc_scatter = pl.kernel(
    _sc_scatter_body,
    out_type=[
        jax.ShapeDtypeStruct((NC, NP, HD), jnp.float32),
        jax.ShapeDtypeStruct((NC, NP, HD), jnp.float32),
    ],
    mesh=plsc.VectorSubcoreMesh(core_axis_name="c", subcore_axis_name="s"),
    scratch_types=[
        pltpu.VMEM((EB,), jnp.int32),
        pltpu.VMEM((EB, HD), jnp.float32),
        pltpu.VMEM((EB, HD), jnp.float32),
        pltpu.VMEM_SHARED((NP, HD), jnp.float32),
        pltpu.VMEM_SHARED((NP, HD), jnp.float32),
    ],
)


def _mm_body(x_ref, w_ref, b_ref, o_ref):
    o_ref[...] = (
        jnp.dot(x_ref[...], w_ref[...], preferred_element_type=jnp.float32)
        + b_ref[...]
    )


def _mm(x, wb, block=512):
    W, b = wb
    n = x.shape[0]
    return pl.pallas_call(
        _mm_body,
        grid=(pl.cdiv(n, block),),
        in_specs=[
            pl.BlockSpec((block, x.shape[1]), lambda i: (i, 0)),
            pl.BlockSpec((x.shape[1], W.shape[1]), lambda i: (0, 0)),
            pl.BlockSpec((1, W.shape[1]), lambda i: (0, 0)),
        ],
        out_specs=pl.BlockSpec((block, W.shape[1]), lambda i: (i, 0)),
        out_shape=jax.ShapeDtypeStruct((n, W.shape[1]), jnp.float32),
    )(x, W, b.reshape(1, -1))


def _k1_body(h_ref, wa, ba, wb, bb, wd, bd, we_, be_, ax_ref, gs_ref, exf_ref):
    h = h_ref[...]
    ax = jnp.dot(h, wa[...], preferred_element_type=jnp.float32) + ba[...]
    bx = jnp.dot(h, wb[...], preferred_element_type=jnp.float32) + bb[...]
    dx = jnp.dot(h, wd[...], preferred_element_type=jnp.float32) + bd[...]
    ex = jnp.dot(h, we_[...], preferred_element_type=jnp.float32) + be_[...]
    ax_ref[...] = ax
    gs_ref[0, :, :] = jnp.concatenate([dx[:, :HD], bx[:, :HD]], axis=1)
    gs_ref[1, :, :] = jnp.concatenate([dx[:, HD:], bx[:, HD:]], axis=1)
    exf_ref[...] = ex


def _k1(h, lp, block=1000):
    args = []
    for nm in ("A", "B", "D", "E"):
        w, b = lp[nm]
        args += [w, b.reshape(1, D)]
    wspec = []
    for _ in range(4):
        wspec += [pl.BlockSpec((D, D), lambda i: (0, 0)),
                  pl.BlockSpec((1, D), lambda i: (0, 0))]
    return pl.pallas_call(
        _k1_body,
        grid=(N // block,),
        in_specs=[pl.BlockSpec((block, D), lambda i: (i, 0))] + wspec,
        out_specs=[
            pl.BlockSpec((block, D), lambda i: (i, 0)),
            pl.BlockSpec((NC, block, D), lambda i: (0, i, 0)),
            pl.BlockSpec((block, D), lambda i: (i, 0)),
        ],
        out_shape=[
            jax.ShapeDtypeStruct((N, D), jnp.float32),
            jax.ShapeDtypeStruct((NC, N, D), jnp.float32),
            jax.ShapeDtypeStruct((N, D), jnp.float32),
        ],
    )(h, *args)


def _ce1_body(e_ref, we_ref, be_ref, wc, bc, ce_ref):
    p = jnp.dot(we_ref[...], wc[...], preferred_element_type=jnp.float32)
    q = jnp.dot(be_ref[...], wc[...], preferred_element_type=jnp.float32) + bc[...]
    ce = e_ref[...] * p + q
    ce_ref[0, :, :] = ce[:, :HD]
    ce_ref[1, :, :] = ce[:, HD:]


def _ce1(E, we, be, wc, bc, block=1280):
    return pl.pallas_call(
        _ce1_body,
        grid=(M // block,),
        in_specs=[
            pl.BlockSpec((block, 1), lambda i: (i, 0)),
            pl.BlockSpec((1, D), lambda i: (0, 0)),
            pl.BlockSpec((1, D), lambda i: (0, 0)),
            pl.BlockSpec((D, D), lambda i: (0, 0)),
            pl.BlockSpec((1, D), lambda i: (0, 0)),
        ],
        out_specs=pl.BlockSpec((NC, block, HD), lambda i: (0, i, 0)),
        out_shape=jax.ShapeDtypeStruct((NC, M, HD), jnp.float32),
    )(E, we.reshape(1, D), be.reshape(1, D), wc, bc.reshape(1, D))


def _ce_body(mode, efp_ref, we_ref, be_ref, u_ref, sc_ref, bi_ref, wc, bc,
             *out):
    if mode == "rank1":
        ef = efp_ref[...] * we_ref[...] + be_ref[...]
    else:
        ef = efp_ref[...]
    u = jnp.concatenate([u_ref[0, :, :], u_ref[1, :, :]], axis=1)
    scl = jnp.concatenate([sc_ref[0, :], sc_ref[1, :]]).reshape(1, D)
    bia = jnp.concatenate([bi_ref[0, :], bi_ref[1, :]]).reshape(1, D)
    en = jnp.maximum(u * scl + bia, 0.0)
    ef = ef + en
    if len(out) == 2:
        out[1][...] = ef
    ce = jnp.dot(ef, wc[...], preferred_element_type=jnp.float32) + bc[...]
    out[0][0, :, :] = ce[:, :HD]
    out[0][1, :, :] = ce[:, HD:]


def _ce_layer(mode, write_ef, efprev, we, be, u, scale, bias, wc, bc,
              block=1280):
    din = 1 if mode == "rank1" else D
    out_specs = [pl.BlockSpec((NC, block, HD), lambda i: (0, i, 0))]
    out_shape = [jax.ShapeDtypeStruct((NC, M, HD), jnp.float32)]
    if write_ef:
        out_specs.append(pl.BlockSpec((block, D), lambda i: (i, 0)))
        out_shape.append(jax.ShapeDtypeStruct((M, D), jnp.float32))
    return pl.pallas_call(
        functools.partial(_ce_body, mode),
        grid=(M // block,),
        in_specs=[
            pl.BlockSpec((block, din), lambda i: (i, 0)),
            pl.BlockSpec((1, D), lambda i: (0, 0)),
            pl.BlockSpec((1, D), lambda i: (0, 0)),
            pl.BlockSpec((NC, block, HD), lambda i: (0, i, 0)),
            pl.BlockSpec((NC, HD), lambda i: (0, 0)),
            pl.BlockSpec((NC, HD), lambda i: (0, 0)),
            pl.BlockSpec((D, D), lambda i: (0, 0)),
            pl.BlockSpec((1, D), lambda i: (0, 0)),
        ],
        out_specs=out_specs,
        out_shape=out_shape,
    )(efprev, we.reshape(1, D), be.reshape(1, D), u, scale, bias,
      wc, bc.reshape(1, D))


def _k3_body(has_stats, ax_ref, num_ref, den_ref, sn_ref, h_ref,
             gh_ref, bh_ref, st_ref, ge_ref, bbe_ref, *out):
    num = num_ref[...]
    den = den_ref[...]
    hn = ax_ref[...] + num / (den + 1e-9)
    hn = hn * sn_ref[...]
    m = jnp.mean(hn, axis=0, keepdims=True)
    v = jnp.mean((hn - m) ** 2, axis=0, keepdims=True)
    hb = gh_ref[...] * (hn - m) / jnp.sqrt(v + 1e-5) + bh_ref[...]
    out[0][...] = h_ref[...] + jnp.maximum(hb, 0.0)
    if has_stats:
        ssum = jnp.sum(st_ref[:, :, :HD], axis=1)
        ssq = jnp.sum(st_ref[:, :, HD:], axis=1)
        me = ssum / M
        ve = ssq / M - me * me
        rstd = 1.0 / jnp.sqrt(ve + 1e-5)
        out[1][...] = ge_ref[...] * rstd
        out[2][...] = bbe_ref[...] - ge_ref[...] * rstd * me


def _k3(ax, num, den, snorm_n, h, lp, stats):
    has_stats = stats is not None
    gh, bh = lp["bnh"]
    ge, bbe = lp["bne"]
    if not has_stats:
        stats = jnp.zeros((NC, NS, D), jnp.float32)
    out_shape = [jax.ShapeDtypeStruct((N, D), jnp.float32)]
    if has_stats:
        out_shape += [jax.ShapeDtypeStruct((NC, HD), jnp.float32),
                      jax.ShapeDtypeStruct((NC, HD), jnp.float32)]
    return pl.pallas_call(
        functools.partial(_k3_body, has_stats),
        out_shape=out_shape,
    )(ax, num, den, snorm_n, h, gh.reshape(1, D), bh.reshape(1, D),
      stats, ge.reshape(NC, HD), bbe.reshape(NC, HD))


def _head_body(h_ref, w1, b1, w2, b2, w3, b3, y_ref):
    y = jnp.mean(h_ref[...], axis=0, keepdims=True)
    y = jnp.maximum(jnp.dot(y, w1[...], preferred_element_type=jnp.float32) + b1[...], 0.0)
    y = jnp.maximum(jnp.dot(y, w2[...], preferred_element_type=jnp.float32) + b2[...], 0.0)
    y_ref[...] = jnp.dot(y, w3[...], preferred_element_type=jnp.float32) + b3[...]


def _head(h, mlp):
    args = []
    for w, b in mlp:
        args += [w, b.reshape(1, -1)]
    return pl.pallas_call(
        _head_body,
        out_shape=jax.ShapeDtypeStruct((1, mlp[-1][0].shape[1]), jnp.float32),
    )(h, *args)


def kernel(X, E, snorm_n, snorm_e, edge_index, params):
    src = edge_index[0]
    dst = edge_index[1]
    srcoff = jnp.concatenate([src, src + N])
    sev = snorm_e[:, 0]
    zrows = jnp.zeros((NPT, HD), jnp.float32)

    H = _mm(X, params["emb_h"])
    we, be = params["emb_e"]
    we = we.reshape(D)

    scale = bias = u_prev = ef = None
    layers = params["layers"]
    for li, lp in enumerate(layers):
        ax, gs, exf = _k1(H, lp)
        gs = gs.reshape(NC * N, D)
        wc, bc = lp["C"]
        if li == 0:
            ce = _ce1(E, we, be, wc, bc)
        elif li == 1:
            ce, ef = _ce_layer("rank1", True, E, we, be, u_prev, scale, bias,
                               wc, bc)
        elif li == 2:
            ce, ef = _ce_layer("full", True, ef, we, be, u_prev, scale, bias,
                               wc, bc)
        else:
            (ce,) = _ce_layer("full", False, ef, we, be, u_prev, scale, bias,
                              wc, bc)
        if li < 3:
            u_prev, nr_rows, sig_rows, stats = _sc_edge_u(
                ce, gs, exf, srcoff, dst, sev)
        else:
            nr_rows, sig_rows = _sc_edge_nou(ce, gs, exf, srcoff, dst, sev)
        nrf = jnp.concatenate([nr_rows[0], nr_rows[1]], axis=1)
        sgf = jnp.concatenate([sig_rows[0], sig_rows[1]], axis=1)
        num = jax.ops.segment_sum(nrf, dst, num_segments=N)
        den = jax.ops.segment_sum(sgf, dst, num_segments=N)
        if li < 3:
            H, scale, bias = _k3(ax, num, den, snorm_n, H, lp,
                                 stats.reshape(NC, NS, D))
        else:
            H = _k3(ax, num, den, snorm_n, H, lp, None)[0]

    return _head(H, params["mlp"])
